# baseline (speedup 1.0000x reference)
import functools

import jax
import jax.numpy as jnp
from jax import lax
from jax.experimental import pallas as pl
from jax.experimental.pallas import tpu as pltpu

_PACK = 16


def _nnconv_kernel(ea_ref, na_ref, w1bd_ref, b1t_ref, w2_ref, b2_ref, root_ref,
                   bias_ref, out_ref, hsum_ref, *, N, HID, IN_C, OUT_C, CH, NC):
    c = pl.program_id(1)
    x = ea_ref[0]
    h = jnp.maximum(
        jnp.dot(x, w1bd_ref[...], preferred_element_type=jnp.float32)
        + b1t_ref[0], 0.0)
    part = jnp.sum(h.reshape(CH // _PACK, _PACK, _PACK * HID), axis=0)

    @pl.when(c == 0)
    def _():
        hsum_ref[...] = part

    @pl.when(c > 0)
    def _():
        hsum_ref[...] = hsum_ref[...] + part

    @pl.when(c == NC - 1)
    def _():
        KW = _PACK * HID
        A = (lax.broadcasted_iota(jnp.int32, (N, _PACK), 0) // _PACK ==
             lax.broadcasted_iota(jnp.int32, (N, _PACK), 1)).astype(jnp.float32)
        G = jnp.dot(A, hsum_ref[...], preferred_element_type=jnp.float32)
        M = (lax.broadcasted_iota(jnp.int32, (N, KW), 1) // HID ==
             lax.broadcasted_iota(jnp.int32, (N, KW), 0) % _PACK).astype(jnp.float32)
        ws = jnp.dot(G * M, w2_ref[...],
                     preferred_element_type=jnp.float32) + N * b2_ref[0]
        na = na_ref[0]
        KO = IN_C * OUT_C
        R = (lax.broadcasted_iota(jnp.int32, (IN_C, KO), 1) // OUT_C ==
             lax.broadcasted_iota(jnp.int32, (IN_C, KO), 0)).astype(jnp.float32)
        S = (lax.broadcasted_iota(jnp.int32, (KO, OUT_C), 0) % OUT_C ==
             lax.broadcasted_iota(jnp.int32, (KO, OUT_C), 1)).astype(jnp.float32)
        msg = jnp.dot(jnp.dot(na, R, preferred_element_type=jnp.float32) * ws, S,
                      preferred_element_type=jnp.float32)
        out_ref[0] = msg + jnp.dot(na, root_ref[...],
                                   preferred_element_type=jnp.float32) + bias_ref[0]


def kernel(node_attr, edge_adj, W1, b1, W2, b2, root, bias):
    B, N, IN_C = node_attr.shape
    D_EDGE = edge_adj.shape[-1]
    HID = W1.shape[1]
    OUT_C = root.shape[1]
    NN = N * N
    ROWS = NN // _PACK
    CH = 1024
    NC = ROWS // CH

    ea_p = edge_adj.reshape(B, ROWS, _PACK * D_EDGE)
    w1bd = jnp.kron(jnp.eye(_PACK, dtype=W1.dtype), W1)
    b1t = jnp.tile(b1, _PACK).reshape(1, _PACK * HID)
    w2t = jnp.tile(W2, (_PACK, 1))
    b2r = b2.reshape(1, IN_C * OUT_C)
    biasr = bias.reshape(1, OUT_C)

    kern = functools.partial(_nnconv_kernel, N=N, HID=HID, IN_C=IN_C,
                             OUT_C=OUT_C, CH=CH, NC=NC)
    out = pl.pallas_call(
        kern,
        grid=(B, NC),
        in_specs=[
            pl.BlockSpec((1, CH, _PACK * D_EDGE), lambda b, c: (b, c, 0)),
            pl.BlockSpec((1, N, IN_C), lambda b, c: (b, 0, 0)),
            pl.BlockSpec((_PACK * D_EDGE, _PACK * HID), lambda b, c: (0, 0)),
            pl.BlockSpec((1, _PACK * HID), lambda b, c: (0, 0)),
            pl.BlockSpec((_PACK * HID, IN_C * OUT_C), lambda b, c: (0, 0)),
            pl.BlockSpec((1, IN_C * OUT_C), lambda b, c: (0, 0)),
            pl.BlockSpec((IN_C, OUT_C), lambda b, c: (0, 0)),
            pl.BlockSpec((1, OUT_C), lambda b, c: (0, 0)),
        ],
        out_specs=pl.BlockSpec((1, N, OUT_C), lambda b, c: (b, 0, 0)),
        out_shape=jax.ShapeDtypeStruct((B, N, OUT_C), jnp.float32),
        scratch_shapes=[pltpu.VMEM((_PACK, _PACK * HID), jnp.float32)],
        compiler_params=pltpu.CompilerParams(
            dimension_semantics=("parallel", "arbitrary")),
    )(ea_p, node_attr, w1bd, b1t, w2t, b2r, root, biasr)
    return out


# speedup vs baseline: 1.8062x; 1.8062x over previous
import functools

import jax
import jax.numpy as jnp
from jax import lax
from jax.experimental import pallas as pl
from jax.experimental.pallas import tpu as pltpu


def _nnconv_kernel(ea0_ref, ea1_ref, na_ref, w1_ref, b1_ref, w2_ref, b2_ref,
                   root_ref, bias_ref, out_ref, hsum_ref, *, N, HID, IN_C,
                   OUT_C, CH, NC):
    c = pl.program_id(1)
    part = None
    for ref in (ea0_ref, ea1_ref):
        x = ref[0]
        h = jnp.maximum(
            jnp.dot(x, w1_ref[...], preferred_element_type=jnp.float32)
            + b1_ref[0], 0.0)
        p = jnp.sum(h.reshape(CH // N, N, HID), axis=0)
        part = p if part is None else part + p

    @pl.when(c == 0)
    def _():
        hsum_ref[...] = part

    @pl.when(c > 0)
    def _():
        hsum_ref[...] = hsum_ref[...] + part

    @pl.when(c == NC - 1)
    def _():
        ws = jnp.dot(hsum_ref[...], w2_ref[...],
                     preferred_element_type=jnp.float32) + N * b2_ref[0]
        na = na_ref[0]
        KO = IN_C * OUT_C
        R = (lax.broadcasted_iota(jnp.int32, (IN_C, KO), 1) // OUT_C ==
             lax.broadcasted_iota(jnp.int32, (IN_C, KO), 0)).astype(jnp.float32)
        S = (lax.broadcasted_iota(jnp.int32, (KO, OUT_C), 0) % OUT_C ==
             lax.broadcasted_iota(jnp.int32, (KO, OUT_C), 1)).astype(jnp.float32)
        msg = jnp.dot(jnp.dot(na, R, preferred_element_type=jnp.float32) * ws, S,
                      preferred_element_type=jnp.float32)
        out_ref[0] = msg + jnp.dot(na, root_ref[...],
                                   preferred_element_type=jnp.float32) + bias_ref[0]


def kernel(node_attr, edge_adj, W1, b1, W2, b2, root, bias):
    B, N, IN_C = node_attr.shape
    D_EDGE = edge_adj.shape[-1]
    HID = W1.shape[1]
    OUT_C = root.shape[1]
    NN = N * N
    CH = 8192
    NC = NN // (2 * CH)

    ea2 = edge_adj.reshape(B, NN, D_EDGE)
    b1r = b1.reshape(1, HID)
    b2r = b2.reshape(1, IN_C * OUT_C)
    biasr = bias.reshape(1, OUT_C)

    kern = functools.partial(_nnconv_kernel, N=N, HID=HID, IN_C=IN_C,
                             OUT_C=OUT_C, CH=CH, NC=NC)
    out = pl.pallas_call(
        kern,
        grid=(B, NC),
        in_specs=[
            pl.BlockSpec((1, CH, D_EDGE), lambda b, c: (b, 2 * c, 0)),
            pl.BlockSpec((1, CH, D_EDGE), lambda b, c: (b, 2 * c + 1, 0)),
            pl.BlockSpec((1, N, IN_C), lambda b, c: (b, 0, 0)),
            pl.BlockSpec((D_EDGE, HID), lambda b, c: (0, 0)),
            pl.BlockSpec((1, HID), lambda b, c: (0, 0)),
            pl.BlockSpec((HID, IN_C * OUT_C), lambda b, c: (0, 0)),
            pl.BlockSpec((1, IN_C * OUT_C), lambda b, c: (0, 0)),
            pl.BlockSpec((IN_C, OUT_C), lambda b, c: (0, 0)),
            pl.BlockSpec((1, OUT_C), lambda b, c: (0, 0)),
        ],
        out_specs=pl.BlockSpec((1, N, OUT_C), lambda b, c: (b, 0, 0)),
        out_shape=jax.ShapeDtypeStruct((B, N, OUT_C), jnp.float32),
        scratch_shapes=[pltpu.VMEM((N, HID), jnp.float32)],
        compiler_params=pltpu.CompilerParams(
            dimension_semantics=("parallel", "arbitrary")),
    )(ea2, ea2, node_attr, W1, b1r, W2, b2r, root, biasr)
    return out
